# baseline (speedup 1.0000x reference)
import jax
import jax.numpy as jnp
from jax.experimental import pallas as pl
from jax.experimental.pallas import tpu as pltpu


def _dot_t(a, w):
    return jax.lax.dot_general(a, w, (((1,), (1,)), ((), ())),
                               preferred_element_type=jnp.float32)


def _mlp_kernel(x_ref, w1_hbm, b1_hbm, w2_hbm, b2_hbm, w3_hbm, b3_hbm, oT_ref,
                w1_v, b1_v, w2_v, b2_v, w3_v, b3_v, sems):
    i = pl.program_id(0)
    pairs = ((w1_hbm, w1_v), (b1_hbm, b1_v), (w2_hbm, w2_v),
             (b2_hbm, b2_v), (w3_hbm, w3_v), (b3_hbm, b3_v))

    @pl.when(i == 0)
    def _():
        for j, (src, dst) in enumerate(pairs):
            pltpu.make_async_copy(src, dst, sems.at[j]).start()
        for j, (src, dst) in enumerate(pairs):
            pltpu.make_async_copy(src, dst, sems.at[j]).wait()

    h1 = jnp.maximum(_dot_t(x_ref[...], w1_v[...]) + b1_v[...], 0.0)
    h2 = jnp.maximum(_dot_t(h1, w2_v[...]) + b2_v[...], 0.0)
    oT = jax.lax.dot_general(w3_v[...], h2, (((1,), (1,)), ((), ())),
                             preferred_element_type=jnp.float32)
    oT_ref[...] = oT + b3_v[...].T


def kernel(x, w1, b1, w2, b2, w3, b3):
    B, F = x.shape
    H1, H2, O = w1.shape[0], w2.shape[0], w3.shape[0]

    TB = min(B, 16384)
    Bp = pl.cdiv(B, TB) * TB
    if Bp != B:
        x = jnp.pad(x, ((0, Bp - B), (0, 0)))

    outT = pl.pallas_call(
        _mlp_kernel,
        out_shape=jax.ShapeDtypeStruct((O, Bp), jnp.float32),
        grid=(Bp // TB,),
        in_specs=[pl.BlockSpec((TB, F), lambda i: (i, 0))] + [
            pl.BlockSpec(memory_space=pl.ANY)] * 6,
        out_specs=pl.BlockSpec((O, TB), lambda i: (0, i)),
        scratch_shapes=[
            pltpu.VMEM((H1, F), jnp.float32),
            pltpu.VMEM((1, H1), jnp.float32),
            pltpu.VMEM((H2, H1), jnp.float32),
            pltpu.VMEM((1, H2), jnp.float32),
            pltpu.VMEM((O, H2), jnp.float32),
            pltpu.VMEM((1, O), jnp.float32),
            pltpu.SemaphoreType.DMA((6,)),
        ],
        compiler_params=pltpu.CompilerParams(
            dimension_semantics=("arbitrary",),
            vmem_limit_bytes=64 * 1024 * 1024,
        ),
        cost_estimate=pl.CostEstimate(
            flops=2 * B * (F * H1 + H1 * H2 + H2 * O),
            transcendentals=0,
            bytes_accessed=4 * (B * F + B * O + F * H1 + H1 + H1 * H2 + H2 + H2 * O + O),
        ),
    )(x, w1, b1.reshape(1, H1), w2, b2.reshape(1, H2), w3, b3.reshape(1, O))

    return outT.T if Bp == B else outT[:, :B].T


# speedup vs baseline: 1.1149x; 1.1149x over previous
import jax
import jax.numpy as jnp
from jax.experimental import pallas as pl
from jax.experimental.pallas import tpu as pltpu


def _dot_t(a, w):
    return jax.lax.dot_general(a, w, (((1,), (1,)), ((), ())),
                               preferred_element_type=jnp.float32)


def _mlp_kernel(x_ref, w1_ref, b1_ref, w2_ref, b2_ref, w3_ref, b3_ref, oT_ref):
    h1 = jnp.maximum(_dot_t(x_ref[...], w1_ref[...]) + b1_ref[...], 0.0)
    h2 = jnp.maximum(_dot_t(h1, w2_ref[...]) + b2_ref[...], 0.0)
    oT = jax.lax.dot_general(w3_ref[...], h2, (((1,), (1,)), ((), ())),
                             preferred_element_type=jnp.float32)
    oT_ref[...] = oT + b3_ref[...].T


def kernel(x, w1, b1, w2, b2, w3, b3):
    B, F = x.shape
    H1, H2, O = w1.shape[0], w2.shape[0], w3.shape[0]

    TB = min(B, 16384)
    Bp = pl.cdiv(B, TB) * TB
    if Bp != B:
        x = jnp.pad(x, ((0, Bp - B), (0, 0)))

    outT = pl.pallas_call(
        _mlp_kernel,
        out_shape=jax.ShapeDtypeStruct((O, Bp), jnp.float32),
        grid=(Bp // TB,),
        in_specs=[
            pl.BlockSpec((TB, F), lambda i: (i, 0)),
            pl.BlockSpec((H1, F), lambda i: (0, 0)),
            pl.BlockSpec((1, H1), lambda i: (0, 0)),
            pl.BlockSpec((H2, H1), lambda i: (0, 0)),
            pl.BlockSpec((1, H2), lambda i: (0, 0)),
            pl.BlockSpec((O, H2), lambda i: (0, 0)),
            pl.BlockSpec((1, O), lambda i: (0, 0)),
        ],
        out_specs=pl.BlockSpec((O, TB), lambda i: (0, i)),
        compiler_params=pltpu.CompilerParams(
            dimension_semantics=("arbitrary",),
            vmem_limit_bytes=64 * 1024 * 1024,
        ),
        cost_estimate=pl.CostEstimate(
            flops=2 * B * (F * H1 + H1 * H2 + H2 * O),
            transcendentals=0,
            bytes_accessed=4 * (B * F + B * O + F * H1 + H1 + H1 * H2 + H2 + H2 * O + O),
        ),
    )(x, w1, b1.reshape(1, H1), w2, b2.reshape(1, H2), w3, b3.reshape(1, O))

    return outT.T if Bp == B else outT[:, :B].T


# speedup vs baseline: 1.1176x; 1.0024x over previous
import jax
import jax.numpy as jnp
from jax.experimental import pallas as pl
from jax.experimental.pallas import tpu as pltpu


def _dot_t(a, w):
    return jax.lax.dot_general(a, w, (((1,), (1,)), ((), ())),
                               preferred_element_type=jnp.float32)


def _mlp_kernel(x_ref, w1_ref, b1_ref, w2_ref, b2_ref, w3_ref, b3_ref, oT_ref):
    h1 = jnp.maximum(_dot_t(x_ref[...], w1_ref[...]) + b1_ref[...], 0.0)
    h2 = jnp.maximum(_dot_t(h1, w2_ref[...]) + b2_ref[...], 0.0)
    oT = jax.lax.dot_general(w3_ref[...], h2, (((1,), (1,)), ((), ())),
                             preferred_element_type=jnp.float32)
    oT_ref[...] = oT + b3_ref[...].T


def kernel(x, w1, b1, w2, b2, w3, b3):
    B, F = x.shape
    H1, H2, O = w1.shape[0], w2.shape[0], w3.shape[0]

    TB = min(B, 16384)
    Bp = pl.cdiv(B, TB) * TB
    if Bp != B:
        x = jnp.pad(x, ((0, Bp - B), (0, 0)))

    outT = pl.pallas_call(
        _mlp_kernel,
        out_shape=jax.ShapeDtypeStruct((O, Bp), jnp.float32),
        grid=(Bp // TB,),
        in_specs=[
            pl.BlockSpec((TB, F), lambda i: (i, 0)),
            pl.BlockSpec((H1, F), lambda i: (0, 0)),
            pl.BlockSpec((1, H1), lambda i: (0, 0)),
            pl.BlockSpec((H2, H1), lambda i: (0, 0)),
            pl.BlockSpec((1, H2), lambda i: (0, 0)),
            pl.BlockSpec((O, H2), lambda i: (0, 0)),
            pl.BlockSpec((1, O), lambda i: (0, 0)),
        ],
        out_specs=pl.BlockSpec((O, TB), lambda i: (0, i)),
        compiler_params=pltpu.CompilerParams(
            dimension_semantics=("parallel",),
            vmem_limit_bytes=64 * 1024 * 1024,
        ),
        cost_estimate=pl.CostEstimate(
            flops=2 * B * (F * H1 + H1 * H2 + H2 * O),
            transcendentals=0,
            bytes_accessed=4 * (B * F + B * O + F * H1 + H1 + H1 * H2 + H2 + H2 * O + O),
        ),
    )(x, w1, b1.reshape(1, H1), w2, b2.reshape(1, H2), w3, b3.reshape(1, O))

    return outT.T if Bp == B else outT[:, :B].T
